# Initial kernel scaffold; baseline (speedup 1.0000x reference)
import jax
import jax.numpy as jnp
from jax.experimental import pallas as pl


def kernel(X, embedding):
    raise NotImplementedError("write your pallas kernel here")


# speedup vs baseline: 1.8031x; 1.8031x over previous
import jax
import jax.numpy as jnp
from jax.experimental import pallas as pl

BATCH = 4
WINDOW = 8192
D_MODEL = 768
BLK_W = 512


def _add_kernel(x_ref, emb_ref, out_ref):
    out_ref[...] = x_ref[...] + emb_ref[...]


def kernel(X, embedding):
    grid = (WINDOW // BLK_W,)
    return pl.pallas_call(
        _add_kernel,
        grid=grid,
        in_specs=[
            pl.BlockSpec((BATCH, BLK_W, D_MODEL), lambda i: (0, i, 0)),
            pl.BlockSpec((BLK_W, D_MODEL), lambda i: (i, 0)),
        ],
        out_specs=pl.BlockSpec((BATCH, BLK_W, D_MODEL), lambda i: (0, i, 0)),
        out_shape=jax.ShapeDtypeStruct((BATCH, WINDOW, D_MODEL), X.dtype),
    )(X, embedding)
